# baseline (speedup 1.0000x reference)
import functools

import jax
import jax.numpy as jnp
from jax import lax
from jax.experimental import pallas as pl
from jax.experimental.pallas import tpu as pltpu
from jax.experimental.pallas import tpu_sc as plsc

N = 10000
E = 320000
D_IN = 128
HID = 64
W = HID + 16

NC = 2
NS = 16
NW = NC * NS
C = 128
NCH = E // C
CHW = NCH // NW
REM = NCH - CHW * NW
CZ = 80
NROWCH = N // CZ
NBUF = 4
LA = NBUF - 1


def _make_edge_scatter():
    grp = W // 16
    mesh = plsc.VectorSubcoreMesh(core_axis_name="c", subcore_axis_name="s")

    @functools.partial(
        pl.kernel,
        out_type=jax.ShapeDtypeStruct((NC, N, W), jnp.float32),
        mesh=mesh,
        scratch_types=[
            pltpu.VMEM_SHARED((N, W), jnp.float32),
            pltpu.VMEM((CHW, C), jnp.int32),
            pltpu.VMEM((CHW, C), jnp.int32),
            pltpu.VMEM((CHW, C), jnp.float32),
            pltpu.VMEM((1, C), jnp.int32),
            pltpu.VMEM((1, C), jnp.int32),
            pltpu.VMEM((1, C), jnp.float32),
            [pltpu.VMEM((C, W), jnp.float32)] * NBUF,
            [pltpu.SemaphoreType.DMA] * NBUF,
            [pltpu.SemaphoreType.DMA] * NBUF,
        ],
        compiler_params=pltpu.CompilerParams(use_tc_tiling_on_sc=False),
    )
    def kern(table, src, dst, ew, out, acc, src_v, dst_v, ew_v,
             src_x, dst_x, ew_x, rows, gsem, ssem):
        c = lax.axis_index("c")
        s = lax.axis_index("s")
        wid = c * NS + s

        sl = pl.ds(wid * CHW, CHW)
        pltpu.sync_copy(src.at[sl], src_v)
        pltpu.sync_copy(dst.at[sl], dst_v)
        pltpu.sync_copy(ew.at[sl], ew_v)

        zeros = jnp.zeros((16,), jnp.float32)

        def zrow(i, carry):
            for g in range(grp):
                rows[0][i, pl.ds(g * 16, 16)] = zeros
            return carry

        lax.fori_loop(0, C, zrow, 0)
        for t in range(-(-NROWCH // NS)):
            j = t * NS + s

            @pl.when(j < NROWCH)
            def _():
                pltpu.sync_copy(rows[0].at[pl.ds(0, CZ)],
                                acc.at[pl.ds(j * CZ, CZ)])

        plsc.subcore_barrier()

        def gather(j, b):
            pltpu.async_copy(table.at[src_v.at[j]], rows[b], gsem[b])

        def gather_wait(j, b):
            pltpu.make_async_copy(table.at[src_v.at[j]], rows[b],
                                  gsem[b]).wait()

        def scatter(j, b):
            pltpu.async_copy(rows[b], acc.at[dst_v.at[j]], ssem[b],
                             add=True)

        def scatter_wait(j, b):
            pltpu.make_async_copy(rows[b], acc.at[dst_v.at[j]],
                                  ssem[b]).wait()

        def scale(b, ew_ref, jrow):
            def grp16(g16, carry):
                w16 = ew_ref[jrow, pl.ds(g16 * 16, 16)]
                for l in range(16):
                    e = g16 * 16 + l
                    w = jnp.broadcast_to(w16[l], (16,))
                    for g in range(grp):
                        rows[b][e, pl.ds(g * 16, 16)] = (
                            rows[b][e, pl.ds(g * 16, 16)] * w)
                return carry

            lax.fori_loop(0, C // 16, grp16, 0)

        def step(j, b):
            nb = (b + LA) % NBUF
            nj = j + LA

            @pl.when(nj < CHW)
            def _():
                @pl.when(j >= 1)
                def _():
                    scatter_wait(j - 1, nb)

                gather(nj, nb)

            gather_wait(j, b)
            scale(b, ew_v, j)
            scatter(j, b)

        for b in range(LA):
            gather(jnp.int32(b), b)

        def hexstep(t, carry):
            for b in range(NBUF):
                step(t * NBUF + b, b)
            return carry

        lax.fori_loop(0, CHW // NBUF, hexstep, 0)
        for b in range(CHW % NBUF):
            step(jnp.int32((CHW // NBUF) * NBUF + b), b)
        for b in range(NBUF):
            j_last = CHW - NBUF + ((b - CHW) % NBUF)
            scatter_wait(jnp.int32(j_last), b)

        @pl.when(wid < REM)
        def _():
            jx = NW * CHW + wid
            pltpu.sync_copy(src.at[pl.ds(jx, 1)], src_x)
            pltpu.sync_copy(dst.at[pl.ds(jx, 1)], dst_x)
            pltpu.sync_copy(ew.at[pl.ds(jx, 1)], ew_x)
            pltpu.async_copy(table.at[src_x.at[0]], rows[0], gsem[0])
            pltpu.make_async_copy(table.at[src_x.at[0]], rows[0],
                                  gsem[0]).wait()
            scale(0, ew_x, 0)
            pltpu.sync_copy(rows[0], acc.at[dst_x.at[0]], add=True)

        plsc.subcore_barrier()
        for t in range(-(-NROWCH // NS)):
            j = t * NS + s

            @pl.when(j < NROWCH)
            def _():
                sl2 = pl.ds(j * CZ, CZ)
                pltpu.sync_copy(acc.at[sl2], out.at[c, sl2])

    return kern


_edge_scatter_l1 = _make_edge_scatter()
_edge_scatter_l2 = _make_edge_scatter()


def _pad_ones(a):
    return jnp.concatenate([a, jnp.ones((a.shape[0], W - HID), jnp.float32)],
                           axis=1)


def _tc1_body(x_ref, we_ref, be_ref, w1_ref, b1_ref, w2_ref, w3_ref, b3_ref,
              ap_ref, bb_ref, cc_ref):
    h = jnp.dot(x_ref[...], we_ref[...],
                preferred_element_type=jnp.float32) + be_ref[...]
    a = jnp.dot(h, w1_ref[...], preferred_element_type=jnp.float32) + b1_ref[...]
    ap_ref[...] = _pad_ones(a)
    bb_ref[...] = jnp.dot(h, w2_ref[...], preferred_element_type=jnp.float32)
    cc_ref[...] = jnp.dot(h, w3_ref[...],
                          preferred_element_type=jnp.float32) + b3_ref[...]


def _tc2_body(p_ref, bb_ref, cc_ref, w1_ref, b1_ref, w2_ref, w3_ref, b3_ref,
              ap_ref, bbs_ref, c1_ref):
    tot = p_ref[0] + p_ref[1]
    agg = tot[:, :HID]
    degw = tot[:, HID:HID + 1]
    h = jnp.maximum(agg - bb_ref[...] * degw + cc_ref[...], 0.0)
    a = jnp.dot(h, w1_ref[...], preferred_element_type=jnp.float32) + b1_ref[...]
    ap_ref[...] = _pad_ones(a)
    bbs_ref[...] = jnp.dot(h, w2_ref[...],
                           preferred_element_type=jnp.float32) * degw
    c1_ref[...] = jnp.dot(h, w3_ref[...],
                          preferred_element_type=jnp.float32) + b3_ref[...]


def _tc3_body(p_ref, bbs_ref, cc_ref, out_ref):
    tot = p_ref[0] + p_ref[1]
    out_ref[...] = jnp.maximum(tot[:, :HID] - bbs_ref[...] + cc_ref[...], 0.0)


def kernel(x, edge_index, edge_attr, batch, W_emb, b_emb,
           W1_0, b1_0, W2_0, W3_0, b3_0,
           W1_1, b1_1, W2_1, W3_1, b3_1):
    del batch
    src = edge_index[0].reshape(NCH, C)
    dst = edge_index[1].reshape(NCH, C)
    ew2 = edge_attr.reshape(NCH, C)

    f32 = jnp.float32
    ap, bb0, c0 = pl.pallas_call(
        _tc1_body,
        out_shape=(
            jax.ShapeDtypeStruct((N, W), f32),
            jax.ShapeDtypeStruct((N, HID), f32),
            jax.ShapeDtypeStruct((N, HID), f32),
        ),
    )(x, W_emb, b_emb.reshape(1, HID), W1_0, b1_0.reshape(1, HID),
      W2_0, W3_0, b3_0.reshape(1, HID))

    p0 = _edge_scatter_l1(ap, src, dst, ew2)

    a1p, bb1s, c1 = pl.pallas_call(
        _tc2_body,
        out_shape=(
            jax.ShapeDtypeStruct((N, W), f32),
            jax.ShapeDtypeStruct((N, HID), f32),
            jax.ShapeDtypeStruct((N, HID), f32),
        ),
    )(p0, bb0, c0, W1_1, b1_1.reshape(1, HID), W2_1, W3_1,
      b3_1.reshape(1, HID))

    p1 = _edge_scatter_l2(a1p, src, dst, ew2)

    h2 = pl.pallas_call(
        _tc3_body,
        out_shape=jax.ShapeDtypeStruct((N, HID), f32),
    )(p1, bb1s, c1)
    return h2


# speedup vs baseline: 1.0147x; 1.0147x over previous
import functools

import jax
import jax.numpy as jnp
from jax import lax
from jax.experimental import pallas as pl
from jax.experimental.pallas import tpu as pltpu
from jax.experimental.pallas import tpu_sc as plsc

N = 10000
E = 320000
D_IN = 128
HID = 64
W = HID + 16

NC = 2
NS = 16
NW = NC * NS
C = 128
NCH = E // C
CHW = NCH // NW
REM = NCH - CHW * NW
CZ = 80
NROWCH = N // CZ
NBUF = 4
LA = NBUF - 1


def _make_edge_scatter():
    grp = W // 16
    mesh = plsc.VectorSubcoreMesh(core_axis_name="c", subcore_axis_name="s")

    @functools.partial(
        pl.kernel,
        out_type=jax.ShapeDtypeStruct((NC, N, W), jnp.float32),
        mesh=mesh,
        scratch_types=[
            pltpu.VMEM_SHARED((N, W), jnp.float32),
            pltpu.VMEM((CHW, C), jnp.int32),
            pltpu.VMEM((2 * CHW, C // 2), jnp.int32),
            pltpu.VMEM((CHW, C), jnp.float32),
            pltpu.VMEM((1, C), jnp.int32),
            pltpu.VMEM((2, C // 2), jnp.int32),
            pltpu.VMEM((1, C), jnp.float32),
            [pltpu.VMEM((C, W), jnp.float32)] * NBUF,
            [pltpu.SemaphoreType.DMA] * NBUF,
            [pltpu.SemaphoreType.DMA] * NBUF,
            [pltpu.SemaphoreType.DMA] * NBUF,
        ],
        compiler_params=pltpu.CompilerParams(use_tc_tiling_on_sc=False),
    )
    def kern(table, src, dst, ew, out, acc, src_v, dst_v, ew_v,
             src_x, dst_x, ew_x, rows, gsem, ssem0, ssem1):
        ssem = (ssem0, ssem1)
        c = lax.axis_index("c")
        s = lax.axis_index("s")
        wid = c * NS + s

        sl = pl.ds(wid * CHW, CHW)
        pltpu.sync_copy(src.at[sl], src_v)
        pltpu.sync_copy(dst.at[pl.ds(wid * 2 * CHW, 2 * CHW)], dst_v)
        pltpu.sync_copy(ew.at[sl], ew_v)

        zeros = jnp.zeros((16,), jnp.float32)

        def zrow(i, carry):
            for g in range(grp):
                rows[0][i, pl.ds(g * 16, 16)] = zeros
            return carry

        lax.fori_loop(0, C, zrow, 0)
        for t in range(-(-NROWCH // NS)):
            j = t * NS + s

            @pl.when(j < NROWCH)
            def _():
                pltpu.sync_copy(rows[0].at[pl.ds(0, CZ)],
                                acc.at[pl.ds(j * CZ, CZ)])

        plsc.subcore_barrier()

        def gather(j, b):
            pltpu.async_copy(table.at[src_v.at[j]], rows[b], gsem[b])

        def gather_wait(j, b):
            pltpu.make_async_copy(table.at[src_v.at[j]], rows[b],
                                  gsem[b]).wait()

        def scatter_half(j, b, h):
            pltpu.async_copy(rows[b].at[pl.ds(h * (C // 2), C // 2)],
                             acc.at[dst_v.at[2 * j + h]], ssem[h][b],
                             add=True)

        def scatter_wait(j, b):
            for h in range(2):
                pltpu.make_async_copy(
                    rows[b].at[pl.ds(h * (C // 2), C // 2)],
                    acc.at[dst_v.at[2 * j + h]], ssem[h][b]).wait()

        def scale_half(b, ew_ref, jrow, h):
            def grp16(g16, carry):
                w16 = ew_ref[jrow, pl.ds(g16 * 16, 16)]
                for l in range(16):
                    e = g16 * 16 + l
                    w = jnp.broadcast_to(w16[l], (16,))
                    for g in range(grp):
                        rows[b][e, pl.ds(g * 16, 16)] = (
                            rows[b][e, pl.ds(g * 16, 16)] * w)
                return carry

            hg = C // 32
            lax.fori_loop(h * hg, (h + 1) * hg, grp16, 0)

        def step(j, b):
            nb = (b + LA) % NBUF
            nj = j + LA

            @pl.when(nj < CHW)
            def _():
                @pl.when(j >= 1)
                def _():
                    scatter_wait(j - 1, nb)

                gather(nj, nb)

            gather_wait(j, b)
            scale_half(b, ew_v, j, 0)
            scatter_half(j, b, 0)
            scale_half(b, ew_v, j, 1)
            scatter_half(j, b, 1)

        for b in range(LA):
            gather(jnp.int32(b), b)

        def hexstep(t, carry):
            for b in range(NBUF):
                step(t * NBUF + b, b)
            return carry

        lax.fori_loop(0, CHW // NBUF, hexstep, 0)
        for b in range(CHW % NBUF):
            step(jnp.int32((CHW // NBUF) * NBUF + b), b)
        for b in range(NBUF):
            j_last = CHW - NBUF + ((b - CHW) % NBUF)
            scatter_wait(jnp.int32(j_last), b)

        @pl.when(wid < REM)
        def _():
            jx = NW * CHW + wid
            pltpu.sync_copy(src.at[pl.ds(jx, 1)], src_x)
            pltpu.sync_copy(dst.at[pl.ds(2 * jx, 2)], dst_x)
            pltpu.sync_copy(ew.at[pl.ds(jx, 1)], ew_x)
            pltpu.async_copy(table.at[src_x.at[0]], rows[0], gsem[0])
            pltpu.make_async_copy(table.at[src_x.at[0]], rows[0],
                                  gsem[0]).wait()
            for h in range(2):
                scale_half(0, ew_x, 0, h)
                pltpu.sync_copy(rows[0].at[pl.ds(h * (C // 2), C // 2)],
                                acc.at[dst_x.at[h]], add=True)

        plsc.subcore_barrier()
        for t in range(-(-NROWCH // NS)):
            j = t * NS + s

            @pl.when(j < NROWCH)
            def _():
                sl2 = pl.ds(j * CZ, CZ)
                pltpu.sync_copy(acc.at[sl2], out.at[c, sl2])

    return kern


_edge_scatter_l1 = _make_edge_scatter()
_edge_scatter_l2 = _make_edge_scatter()


def _pad_ones(a):
    return jnp.concatenate([a, jnp.ones((a.shape[0], W - HID), jnp.float32)],
                           axis=1)


def _tc1_body(x_ref, we_ref, be_ref, w1_ref, b1_ref, w2_ref, w3_ref, b3_ref,
              ap_ref, bb_ref, cc_ref):
    h = jnp.dot(x_ref[...], we_ref[...],
                preferred_element_type=jnp.float32) + be_ref[...]
    a = jnp.dot(h, w1_ref[...], preferred_element_type=jnp.float32) + b1_ref[...]
    ap_ref[...] = _pad_ones(a)
    bb_ref[...] = jnp.dot(h, w2_ref[...], preferred_element_type=jnp.float32)
    cc_ref[...] = jnp.dot(h, w3_ref[...],
                          preferred_element_type=jnp.float32) + b3_ref[...]


def _tc2_body(p_ref, bb_ref, cc_ref, w1_ref, b1_ref, w2_ref, w3_ref, b3_ref,
              ap_ref, bbs_ref, c1_ref):
    tot = p_ref[0] + p_ref[1]
    agg = tot[:, :HID]
    degw = tot[:, HID:HID + 1]
    h = jnp.maximum(agg - bb_ref[...] * degw + cc_ref[...], 0.0)
    a = jnp.dot(h, w1_ref[...], preferred_element_type=jnp.float32) + b1_ref[...]
    ap_ref[...] = _pad_ones(a)
    bbs_ref[...] = jnp.dot(h, w2_ref[...],
                           preferred_element_type=jnp.float32) * degw
    c1_ref[...] = jnp.dot(h, w3_ref[...],
                          preferred_element_type=jnp.float32) + b3_ref[...]


def _tc3_body(p_ref, bbs_ref, cc_ref, out_ref):
    tot = p_ref[0] + p_ref[1]
    out_ref[...] = jnp.maximum(tot[:, :HID] - bbs_ref[...] + cc_ref[...], 0.0)


def kernel(x, edge_index, edge_attr, batch, W_emb, b_emb,
           W1_0, b1_0, W2_0, W3_0, b3_0,
           W1_1, b1_1, W2_1, W3_1, b3_1):
    del batch
    src = edge_index[0].reshape(NCH, C)
    dst = edge_index[1].reshape(2 * NCH, C // 2)
    ew2 = edge_attr.reshape(NCH, C)

    f32 = jnp.float32
    ap, bb0, c0 = pl.pallas_call(
        _tc1_body,
        out_shape=(
            jax.ShapeDtypeStruct((N, W), f32),
            jax.ShapeDtypeStruct((N, HID), f32),
            jax.ShapeDtypeStruct((N, HID), f32),
        ),
    )(x, W_emb, b_emb.reshape(1, HID), W1_0, b1_0.reshape(1, HID),
      W2_0, W3_0, b3_0.reshape(1, HID))

    p0 = _edge_scatter_l1(ap, src, dst, ew2)

    a1p, bb1s, c1 = pl.pallas_call(
        _tc2_body,
        out_shape=(
            jax.ShapeDtypeStruct((N, W), f32),
            jax.ShapeDtypeStruct((N, HID), f32),
            jax.ShapeDtypeStruct((N, HID), f32),
        ),
    )(p0, bb0, c0, W1_1, b1_1.reshape(1, HID), W2_1, W3_1,
      b3_1.reshape(1, HID))

    p1 = _edge_scatter_l2(a1p, src, dst, ew2)

    h2 = pl.pallas_call(
        _tc3_body,
        out_shape=jax.ShapeDtypeStruct((N, HID), f32),
    )(p1, bb1s, c1)
    return h2


# speedup vs baseline: 1.0359x; 1.0210x over previous
import functools

import jax
import jax.numpy as jnp
from jax import lax
from jax.experimental import pallas as pl
from jax.experimental.pallas import tpu as pltpu
from jax.experimental.pallas import tpu_sc as plsc

N = 10000
E = 320000
D_IN = 128
HID = 64
W = HID + 16

NC = 2
NS = 16
NW = NC * NS
C = 128
NCH = E // C
CHW = NCH // NW
REM = NCH - CHW * NW
CZ = 80
NROWCH = N // CZ
NBUF = 4
LA = NBUF - 1


def _make_edge_scatter():
    grp = W // 16
    mesh = plsc.VectorSubcoreMesh(core_axis_name="c", subcore_axis_name="s")

    @functools.partial(
        pl.kernel,
        out_type=jax.ShapeDtypeStruct((NC, N, W), jnp.float32),
        mesh=mesh,
        scratch_types=[
            pltpu.VMEM_SHARED((N, W), jnp.float32),
            pltpu.VMEM((CHW, C), jnp.int32),
            pltpu.VMEM((2 * CHW, C // 2), jnp.int32),
            pltpu.VMEM((CHW, C), jnp.float32),
            pltpu.VMEM((1, C), jnp.int32),
            pltpu.VMEM((2, C // 2), jnp.int32),
            pltpu.VMEM((1, C), jnp.float32),
            [pltpu.VMEM((C, W), jnp.float32)] * NBUF,
            [pltpu.SemaphoreType.DMA] * NBUF,
            [pltpu.SemaphoreType.DMA] * NBUF,
            [pltpu.SemaphoreType.DMA] * NBUF,
        ],
        compiler_params=pltpu.CompilerParams(use_tc_tiling_on_sc=False),
    )
    def kern(table, src, dst, ew, out, acc, src_v, dst_v, ew_v,
             src_x, dst_x, ew_x, rows, gsem, ssem0, ssem1):
        ssem = (ssem0, ssem1)
        c = lax.axis_index("c")
        s = lax.axis_index("s")
        wid = c * NS + s

        sl = pl.ds(wid * CHW, CHW)
        pltpu.sync_copy(src.at[sl], src_v)
        pltpu.sync_copy(dst.at[pl.ds(wid * 2 * CHW, 2 * CHW)], dst_v)
        pltpu.sync_copy(ew.at[sl], ew_v)

        def gather(j, b):
            pltpu.async_copy(table.at[src_v.at[j]], rows[b], gsem[b])

        def gather_wait(j, b):
            pltpu.make_async_copy(table.at[src_v.at[j]], rows[b],
                                  gsem[b]).wait()

        def scatter_half(j, b, h):
            pltpu.async_copy(rows[b].at[pl.ds(h * (C // 2), C // 2)],
                             acc.at[dst_v.at[2 * j + h]], ssem[h][b],
                             add=True)

        def scatter_wait(j, b):
            for h in range(2):
                pltpu.make_async_copy(
                    rows[b].at[pl.ds(h * (C // 2), C // 2)],
                    acc.at[dst_v.at[2 * j + h]], ssem[h][b]).wait()

        def scale_half(b, ew_ref, jrow, h):
            def grp16(g16, carry):
                w16 = ew_ref[jrow, pl.ds(g16 * 16, 16)]
                for l in range(16):
                    e = g16 * 16 + l
                    w = jnp.broadcast_to(w16[l], (16,))
                    for g in range(grp):
                        rows[b][e, pl.ds(g * 16, 16)] = (
                            rows[b][e, pl.ds(g * 16, 16)] * w)
                return carry

            hg = C // 32
            lax.fori_loop(h * hg, (h + 1) * hg, grp16, 0)

        def step(j, b):
            nb = (b + LA) % NBUF
            nj = j + LA

            @pl.when(nj < CHW)
            def _():
                @pl.when(j >= 1)
                def _():
                    scatter_wait(j - 1, nb)

                gather(nj, nb)

            gather_wait(j, b)
            scale_half(b, ew_v, j, 0)
            scatter_half(j, b, 0)
            scale_half(b, ew_v, j, 1)
            scatter_half(j, b, 1)

        for b in range(LA):
            gather(jnp.int32(b), b)

        zeros = jnp.zeros((16,), jnp.float32)

        def zrow(i, carry):
            for g in range(grp):
                rows[LA][i, pl.ds(g * 16, 16)] = zeros
            return carry

        lax.fori_loop(0, CZ, zrow, 0)
        for t in range(-(-NROWCH // NS)):
            j = t * NS + s

            @pl.when(j < NROWCH)
            def _():
                pltpu.sync_copy(rows[LA].at[pl.ds(0, CZ)],
                                acc.at[pl.ds(j * CZ, CZ)])

        plsc.subcore_barrier()

        def hexstep(t, carry):
            for b in range(NBUF):
                step(t * NBUF + b, b)
            return carry

        lax.fori_loop(0, CHW // NBUF, hexstep, 0)
        for b in range(CHW % NBUF):
            step(jnp.int32((CHW // NBUF) * NBUF + b), b)
        for b in range(NBUF):
            j_last = CHW - NBUF + ((b - CHW) % NBUF)
            scatter_wait(jnp.int32(j_last), b)

        @pl.when(wid < REM)
        def _():
            jx = NW * CHW + wid
            pltpu.sync_copy(src.at[pl.ds(jx, 1)], src_x)
            pltpu.sync_copy(dst.at[pl.ds(2 * jx, 2)], dst_x)
            pltpu.sync_copy(ew.at[pl.ds(jx, 1)], ew_x)
            pltpu.async_copy(table.at[src_x.at[0]], rows[0], gsem[0])
            pltpu.make_async_copy(table.at[src_x.at[0]], rows[0],
                                  gsem[0]).wait()
            for h in range(2):
                scale_half(0, ew_x, 0, h)
                pltpu.sync_copy(rows[0].at[pl.ds(h * (C // 2), C // 2)],
                                acc.at[dst_x.at[h]], add=True)

        plsc.subcore_barrier()
        for t in range(-(-NROWCH // NS)):
            j = t * NS + s

            @pl.when(j < NROWCH)
            def _():
                sl2 = pl.ds(j * CZ, CZ)
                pltpu.sync_copy(acc.at[sl2], out.at[c, sl2])

    return kern


_edge_scatter_l1 = _make_edge_scatter()
_edge_scatter_l2 = _make_edge_scatter()


def _pad_ones(a):
    return jnp.concatenate([a, jnp.ones((a.shape[0], W - HID), jnp.float32)],
                           axis=1)


def _tc1_body(x_ref, we_ref, be_ref, w1_ref, b1_ref, w2_ref, w3_ref, b3_ref,
              ap_ref, bb_ref, cc_ref):
    h = jnp.dot(x_ref[...], we_ref[...],
                preferred_element_type=jnp.float32) + be_ref[...]
    a = jnp.dot(h, w1_ref[...], preferred_element_type=jnp.float32) + b1_ref[...]
    ap_ref[...] = _pad_ones(a)
    bb_ref[...] = jnp.dot(h, w2_ref[...], preferred_element_type=jnp.float32)
    cc_ref[...] = jnp.dot(h, w3_ref[...],
                          preferred_element_type=jnp.float32) + b3_ref[...]


def _tc2_body(p_ref, bb_ref, cc_ref, w1_ref, b1_ref, w2_ref, w3_ref, b3_ref,
              ap_ref, bbs_ref, c1_ref):
    tot = p_ref[0] + p_ref[1]
    agg = tot[:, :HID]
    degw = tot[:, HID:HID + 1]
    h = jnp.maximum(agg - bb_ref[...] * degw + cc_ref[...], 0.0)
    a = jnp.dot(h, w1_ref[...], preferred_element_type=jnp.float32) + b1_ref[...]
    ap_ref[...] = _pad_ones(a)
    bbs_ref[...] = jnp.dot(h, w2_ref[...],
                           preferred_element_type=jnp.float32) * degw
    c1_ref[...] = jnp.dot(h, w3_ref[...],
                          preferred_element_type=jnp.float32) + b3_ref[...]


def _tc3_body(p_ref, bbs_ref, cc_ref, out_ref):
    tot = p_ref[0] + p_ref[1]
    out_ref[...] = jnp.maximum(tot[:, :HID] - bbs_ref[...] + cc_ref[...], 0.0)


def kernel(x, edge_index, edge_attr, batch, W_emb, b_emb,
           W1_0, b1_0, W2_0, W3_0, b3_0,
           W1_1, b1_1, W2_1, W3_1, b3_1):
    del batch
    src = edge_index[0].reshape(NCH, C)
    dst = edge_index[1].reshape(2 * NCH, C // 2)
    ew2 = edge_attr.reshape(NCH, C)

    f32 = jnp.float32
    ap, bb0, c0 = pl.pallas_call(
        _tc1_body,
        out_shape=(
            jax.ShapeDtypeStruct((N, W), f32),
            jax.ShapeDtypeStruct((N, HID), f32),
            jax.ShapeDtypeStruct((N, HID), f32),
        ),
    )(x, W_emb, b_emb.reshape(1, HID), W1_0, b1_0.reshape(1, HID),
      W2_0, W3_0, b3_0.reshape(1, HID))

    p0 = _edge_scatter_l1(ap, src, dst, ew2)

    a1p, bb1s, c1 = pl.pallas_call(
        _tc2_body,
        out_shape=(
            jax.ShapeDtypeStruct((N, W), f32),
            jax.ShapeDtypeStruct((N, HID), f32),
            jax.ShapeDtypeStruct((N, HID), f32),
        ),
    )(p0, bb0, c0, W1_1, b1_1.reshape(1, HID), W2_1, W3_1,
      b3_1.reshape(1, HID))

    p1 = _edge_scatter_l2(a1p, src, dst, ew2)

    h2 = pl.pallas_call(
        _tc3_body,
        out_shape=jax.ShapeDtypeStruct((N, HID), f32),
    )(p1, bb1s, c1)
    return h2
